# baseline (speedup 1.0000x reference)
import functools

import jax
import jax.numpy as jnp
from jax import lax
from jax.experimental import pallas as pl
from jax.experimental.pallas import tpu as pltpu
from jax.experimental.pallas import tpu_sc as plsc

R, C = 16384, 200
D = 32
N = R * C
NW = 32
B = 1600
IPW = N // NW
NCH = IPW // B
L = 16
SB = 320


def _sc_lookup(labels_2d, table_flat):
    mesh = plsc.VectorSubcoreMesh(core_axis_name="c", subcore_axis_name="s")

    @functools.partial(
        pl.kernel,
        mesh=mesh,
        out_type=jax.ShapeDtypeStruct((N // B, B * D), jnp.float32),
        scratch_types=[
            pltpu.VMEM((128,), jnp.float32),
            pltpu.VMEM((B,), jnp.int32),
            pltpu.VMEM((B,), jnp.int32),
            pltpu.VMEM((B * D,), jnp.float32),
            pltpu.VMEM((B * D,), jnp.float32),
            pltpu.SemaphoreType.DMA,
            pltpu.SemaphoreType.DMA,
            pltpu.SemaphoreType.DMA,
            pltpu.SemaphoreType.DMA,
        ],
    )
    def k(labels_hbm, table_hbm, out_hbm, tab_v, idx0, idx1,
          rows0, rows1, semi0, semi1, semo0, semo1):
        idx_b = (idx0, idx1)
        rows_b = (rows0, rows1)
        semi_b = (semi0, semi1)
        semo_b = (semo0, semo1)
        wid = lax.axis_index("s") * 2 + lax.axis_index("c")
        base = wid * NCH
        pltpu.sync_copy(table_hbm, tab_v)
        t1lo = tab_v[pl.ds(D, L)]
        t1hi = tab_v[pl.ds(D + L, L)]
        t2lo = tab_v[pl.ds(2 * D, L)]
        t2hi = tab_v[pl.ds(2 * D + L, L)]
        zero = jnp.zeros((L,), jnp.float32)
        lane = [jnp.full((L, 1), u, jnp.int32) for u in range(L)]
        dnums = lax.GatherDimensionNumbers(
            offset_dims=(), collapsed_slice_dims=(0,), start_index_map=(0,)
        )

        def bcast_lane(vec, u):
            return lax.gather(
                vec, lane[u], dnums, slice_sizes=(1,),
                mode=lax.GatherScatterMode.PROMISE_IN_BOUNDS,
            )

        def expand(idx_v, rows_v):
            def group(j, carry):
                iv = idx_v[pl.ds(j * L, L)]
                f1 = (iv * (2 - iv)).astype(jnp.float32)
                f2 = ((iv * (iv - 1)) >> 1).astype(jnp.float32)
                rb = j * (L * D)
                for u in range(L):
                    w1 = bcast_lane(f1, u)
                    w2 = bcast_lane(f2, u)
                    rows_v[pl.ds(rb + u * D, L)] = w1 * t1lo + w2 * t2lo
                    rows_v[pl.ds(rb + u * D + L, L)] = w1 * t1hi + w2 * t2hi
                return carry

            lax.fori_loop(0, B // L, group, 0)

        pltpu.async_copy(labels_hbm.at[base], idx0, semi0)
        pltpu.async_copy(labels_hbm.at[base + 1], idx1, semi1)

        def pair(i, carry):
            for b in range(2):
                c = base + i * 2 + b
                pltpu.make_async_copy(labels_hbm.at[c], idx_b[b], semi_b[b]).wait()

                @pl.when(i > 0)
                def _wait_out():
                    pltpu.make_async_copy(rows_b[b], out_hbm.at[c], semo_b[b]).wait()

                expand(idx_b[b], rows_b[b])
                pltpu.async_copy(rows_b[b], out_hbm.at[c], semo_b[b])

                @pl.when(i < NCH // 2 - 1)
                def _prefetch():
                    pltpu.async_copy(labels_hbm.at[c + 2], idx_b[b], semi_b[b])
            return carry

        lax.fori_loop(0, NCH // 2, pair, 0)

        for b in range(2):
            pltpu.make_async_copy(rows_b[b], out_hbm.at[0], semo_b[b]).wait()

    return k(labels_2d, table_flat)


def kernel(speaker_labels, table):
    t = jnp.zeros((128,), jnp.float32).at[: 3 * D].set(
        table.at[0].set(0.0).reshape(-1)
    )
    labels = speaker_labels.astype(jnp.int32).reshape(N // B, B)
    out = _sc_lookup(labels, t)
    return out.reshape(R, C, D)


# speedup vs baseline: 3.5927x; 3.5927x over previous
import functools

import jax
import jax.numpy as jnp
from jax import lax
from jax.experimental import pallas as pl
from jax.experimental.pallas import tpu as pltpu
from jax.experimental.pallas import tpu_sc as plsc

R, C = 16384, 200
D = 32
NW = 32
BI = 256
NBLK = R // BI
BPW = NBLK // NW
L = 16
NG = BI // L


def _sc_lookup(labels_t, tabx):
    mesh = plsc.VectorSubcoreMesh(core_axis_name="c", subcore_axis_name="s")

    @functools.partial(
        pl.kernel,
        mesh=mesh,
        out_type=jax.ShapeDtypeStruct((C, D, R), jnp.float32),
        scratch_types=[
            pltpu.VMEM((2 * D * L,), jnp.float32),
            pltpu.VMEM((C, BI), jnp.int32),
            pltpu.VMEM((D, BI), jnp.float32),
            pltpu.VMEM((D, BI), jnp.float32),
            pltpu.SemaphoreType.DMA,
            pltpu.SemaphoreType.DMA,
        ],
    )
    def k(labels_hbm, tabx_hbm, out_hbm, tabx_v, slab_v, tile0, tile1,
          semo0, semo1):
        tile_b = (tile0, tile1)
        semo_b = (semo0, semo1)
        wid = lax.axis_index("s") * 2 + lax.axis_index("c")
        pltpu.sync_copy(tabx_hbm, tabx_v)

        def jrow(slab_v, tile_v, j):
            for g in range(NG):
                lv = slab_v[j, pl.ds(g * L, L)]
                w1 = (lv * (2 - lv)).astype(jnp.float32)
                w2 = ((lv * (lv - 1)) >> 1).astype(jnp.float32)
                for d in range(D):
                    t1d = tabx_v[pl.ds(d * L, L)]
                    t2d = tabx_v[pl.ds((D + d) * L, L)]
                    tile_v[d, pl.ds(g * L, L)] = w1 * t1d + w2 * t2d

        for blk in range(BPW):
            i0 = (wid * BPW + blk) * BI
            pltpu.sync_copy(labels_hbm.at[:, pl.ds(i0, BI)], slab_v)

            def pair(p, carry):
                for b in range(2):
                    j = p * 2 + b

                    @pl.when(p > 0)
                    def _wait_out():
                        pltpu.make_async_copy(
                            tile_b[b], out_hbm.at[0, :, pl.ds(i0, BI)],
                            semo_b[b],
                        ).wait()

                    jrow(slab_v, tile_b[b], j)
                    pltpu.async_copy(
                        tile_b[b], out_hbm.at[j, :, pl.ds(i0, BI)], semo_b[b]
                    )
                return carry

            lax.fori_loop(0, C // 2, pair, 0)
            for b in range(2):
                pltpu.make_async_copy(
                    tile_b[b], out_hbm.at[0, :, pl.ds(i0, BI)], semo_b[b]
                ).wait()

    return k(labels_t, tabx)


def kernel(speaker_labels, table):
    t = table.at[0].set(0.0)
    tabx = jnp.broadcast_to(t[1:3].reshape(2, D, 1), (2, D, L)).reshape(-1)
    labels_t = speaker_labels.astype(jnp.int32).T
    out = _sc_lookup(labels_t, tabx)
    return out.transpose(2, 0, 1)


# speedup vs baseline: 5.3788x; 1.4971x over previous
import functools

import jax
import jax.numpy as jnp
from jax import lax
from jax.experimental import pallas as pl
from jax.experimental.pallas import tpu as pltpu
from jax.experimental.pallas import tpu_sc as plsc

R, C = 16384, 200
D = 32
NW = 32
BI = 256
NBLK = R // BI
BPW = NBLK // NW
L = 16
NG = BI // L
GQ = 4
JB = 4
NQ = C // JB


def _sc_lookup(labels_t, tabx):
    mesh = plsc.VectorSubcoreMesh(core_axis_name="c", subcore_axis_name="s")

    @functools.partial(
        pl.kernel,
        mesh=mesh,
        out_type=jax.ShapeDtypeStruct((C, D, R), jnp.float32),
        scratch_types=[
            pltpu.VMEM((2 * D * L,), jnp.float32),
            pltpu.VMEM((C, BI), jnp.int32),
            pltpu.VMEM((JB, D, BI), jnp.float32),
            pltpu.VMEM((JB, D, BI), jnp.float32),
            pltpu.SemaphoreType.DMA,
            pltpu.SemaphoreType.DMA,
        ],
    )
    def k(labels_hbm, tabx_hbm, out_hbm, tabx_v, slab_v, quad0, quad1,
          semo0, semo1):
        quad_b = (quad0, quad1)
        semo_b = (semo0, semo1)
        wid = lax.axis_index("s") * 2 + lax.axis_index("c")
        pltpu.sync_copy(tabx_hbm, tabx_v)

        def jrow(quad_v, jj, j):
            def gquad(gq, carry):
                g0 = gq * GQ
                w1s, w2s = [], []
                for u in range(GQ):
                    lv = slab_v[j, pl.ds((g0 + u) * L, L)]
                    w1s.append((lv * (2 - lv)).astype(jnp.float32))
                    w2s.append(((lv * (lv - 1)) >> 1).astype(jnp.float32))
                for d in range(D):
                    t1d = tabx_v[pl.ds(d * L, L)]
                    t2d = tabx_v[pl.ds((D + d) * L, L)]
                    for u in range(GQ):
                        quad_v[jj, d, pl.ds((g0 + u) * L, L)] = (
                            w1s[u] * t1d + w2s[u] * t2d
                        )
                return carry

            lax.fori_loop(0, NG // GQ, gquad, 0)

        for blk in range(BPW):
            i0 = (wid * BPW + blk) * BI
            pltpu.sync_copy(labels_hbm.at[:, pl.ds(i0, BI)], slab_v)

            def pair(p, carry):
                for b in range(2):
                    q = p * 2 + b
                    j0 = q * JB

                    @pl.when(p > 0)
                    def _wait_out():
                        pltpu.make_async_copy(
                            quad_b[b],
                            out_hbm.at[pl.ds(0, JB), :, pl.ds(i0, BI)],
                            semo_b[b],
                        ).wait()

                    def jbody(jj, cc):
                        jrow(quad_b[b], jj, j0 + jj)
                        return cc

                    lax.fori_loop(0, JB, jbody, 0)
                    pltpu.async_copy(
                        quad_b[b],
                        out_hbm.at[pl.ds(j0, JB), :, pl.ds(i0, BI)],
                        semo_b[b],
                    )
                return carry

            lax.fori_loop(0, NQ // 2, pair, 0)
            for b in range(2):
                pltpu.make_async_copy(
                    quad_b[b],
                    out_hbm.at[pl.ds(0, JB), :, pl.ds(i0, BI)],
                    semo_b[b],
                ).wait()

    return k(labels_t, tabx)


def kernel(speaker_labels, table):
    t = table.at[0].set(0.0)
    tabx = jnp.broadcast_to(t[1:3].reshape(2, D, 1), (2, D, L)).reshape(-1)
    labels_t = speaker_labels.astype(jnp.int32).T
    out = _sc_lookup(labels_t, tabx)
    return out.transpose(2, 0, 1)


# speedup vs baseline: 9.5206x; 1.7700x over previous
import functools

import jax
import jax.numpy as jnp
from jax import lax
from jax.experimental import pallas as pl
from jax.experimental.pallas import tpu as pltpu
from jax.experimental.pallas import tpu_sc as plsc

R, C = 16384, 200
D = 32
NW = 32
BI = 256
NBLK = R // BI
BPW = NBLK // NW
L = 16
NG = BI // L
GQ = 8
JB = 4
NQ = C // JB


def _sc_lookup(labels_t, tabx):
    mesh = plsc.VectorSubcoreMesh(core_axis_name="c", subcore_axis_name="s")

    @functools.partial(
        pl.kernel,
        mesh=mesh,
        out_type=jax.ShapeDtypeStruct((C, D, R), jnp.float32),
        scratch_types=[
            pltpu.VMEM((D * L,), jnp.float32),
            pltpu.VMEM((C, BI), jnp.int32),
            pltpu.VMEM((JB, D, BI), jnp.float32),
            pltpu.VMEM((JB, D, BI), jnp.float32),
            pltpu.SemaphoreType.DMA,
            pltpu.SemaphoreType.DMA,
        ],
    )
    def k(labels_hbm, tabx_hbm, out_hbm, tabx_v, slab_v, quad0, quad1,
          semo0, semo1):
        quad_b = (quad0, quad1)
        semo_b = (semo0, semo1)
        wid = lax.axis_index("s") * 2 + lax.axis_index("c")
        pltpu.sync_copy(tabx_hbm, tabx_v)
        dnums = lax.GatherDimensionNumbers(
            offset_dims=(), collapsed_slice_dims=(0,), start_index_map=(0,)
        )

        def pick(vd, lv):
            return lax.gather(
                vd, lv[:, None], dnums, slice_sizes=(1,),
                mode=lax.GatherScatterMode.PROMISE_IN_BOUNDS,
            )

        def jrow(quad_v, jj, j):
            def gquad(gq, carry):
                g0 = gq * GQ
                lvs = [
                    slab_v[j, pl.ds((g0 + u) * L, L)] for u in range(GQ)
                ]
                for d in range(D):
                    vd = tabx_v[pl.ds(d * L, L)]
                    for u in range(GQ):
                        quad_v[jj, d, pl.ds((g0 + u) * L, L)] = (
                            pick(vd, lvs[u])
                        )
                return carry

            lax.fori_loop(0, NG // GQ, gquad, 0)

        for blk in range(BPW):
            i0 = (wid * BPW + blk) * BI
            pltpu.sync_copy(labels_hbm.at[:, pl.ds(i0, BI)], slab_v)

            def pair(p, carry):
                for b in range(2):
                    q = p * 2 + b
                    j0 = q * JB

                    @pl.when(p > 0)
                    def _wait_out():
                        pltpu.make_async_copy(
                            quad_b[b],
                            out_hbm.at[pl.ds(0, JB), :, pl.ds(i0, BI)],
                            semo_b[b],
                        ).wait()

                    def jbody(jj, cc):
                        jrow(quad_b[b], jj, j0 + jj)
                        return cc

                    lax.fori_loop(0, JB, jbody, 0)
                    pltpu.async_copy(
                        quad_b[b],
                        out_hbm.at[pl.ds(j0, JB), :, pl.ds(i0, BI)],
                        semo_b[b],
                    )
                return carry

            lax.fori_loop(0, NQ // 2, pair, 0)
            for b in range(2):
                pltpu.make_async_copy(
                    quad_b[b],
                    out_hbm.at[pl.ds(0, JB), :, pl.ds(i0, BI)],
                    semo_b[b],
                ).wait()

    return k(labels_t, tabx)


def kernel(speaker_labels, table):
    t = table.at[0].set(0.0)
    tabx = jnp.zeros((D, L), jnp.float32).at[:, :3].set(t.T).reshape(-1)
    labels_t = speaker_labels.astype(jnp.int32).T
    out = _sc_lookup(labels_t, tabx)
    return out.transpose(2, 0, 1)
